# baseline (speedup 1.0000x reference)
import functools

import jax
import jax.numpy as jnp
from jax import lax
from jax.experimental import pallas as pl
from jax.experimental.pallas import tpu as pltpu

B = 1
S = 2048
REL_NUM = 16
HEAD = 64
ROWS_PER_BLOCK = 8


def _fanout_body(e_sm, e_col_ref, w_ref, out_ref, slab_ref):
    i = pl.program_id(0)

    @pl.when(i == 0)
    def _build_slabs():
        e_col = e_col_ref[...]
        iota_r = lax.broadcasted_iota(jnp.int32, (S, REL_NUM), 1)
        w = w_ref[...]
        for a in range(4):
            onehot = (iota_r == (e_col + 4 * a)).astype(jnp.float32)
            slab_ref[a] = jnp.dot(
                onehot, w, preferred_element_type=jnp.float32
            )

    for k in range(ROWS_PER_BLOCK):
        a_k = e_sm[i * ROWS_PER_BLOCK + k]
        out_ref[0, k] = slab_ref[a_k]


@jax.jit
def kernel(evidence_type, W):
    e = evidence_type.reshape(S).astype(jnp.int32)
    e_col = e.reshape(S, 1)

    grid_spec = pltpu.PrefetchScalarGridSpec(
        num_scalar_prefetch=1,
        grid=(S // ROWS_PER_BLOCK,),
        in_specs=[
            pl.BlockSpec((S, 1), lambda i, e_sm: (0, 0)),
            pl.BlockSpec((REL_NUM, HEAD), lambda i, e_sm: (0, 0)),
        ],
        out_specs=pl.BlockSpec(
            (1, ROWS_PER_BLOCK, S, HEAD), lambda i, e_sm: (0, i, 0, 0)
        ),
        scratch_shapes=[pltpu.VMEM((4, S, HEAD), jnp.float32)],
    )

    out = pl.pallas_call(
        _fanout_body,
        grid_spec=grid_spec,
        out_shape=jax.ShapeDtypeStruct((B, S, S, HEAD), jnp.float32),
    )(e, e_col, W)
    return out


# speedup vs baseline: 1.0017x; 1.0017x over previous
import functools

import jax
import jax.numpy as jnp
from jax import lax
from jax.experimental import pallas as pl
from jax.experimental.pallas import tpu as pltpu

B = 1
S = 2048
REL_NUM = 16
HEAD = 64
ROWS_PER_BLOCK = 8


def _fanout_body(e_sm, e_col_ref, w_ref, out_ref, slab_ref):
    i = pl.program_id(0)

    @pl.when(i == 0)
    def _build_slabs():
        e_col = e_col_ref[...]
        iota_r = lax.broadcasted_iota(jnp.int32, (S, REL_NUM), 1)
        w = w_ref[...]
        for a in range(4):
            onehot = (iota_r == (e_col + 4 * a)).astype(jnp.float32)
            slab_ref[a] = jnp.dot(
                onehot, w, preferred_element_type=jnp.float32
            )

    for k in range(ROWS_PER_BLOCK):
        a_k = e_sm[i * ROWS_PER_BLOCK + k]
        out_ref[k] = slab_ref[a_k]


@jax.jit
def kernel(evidence_type, W):
    e = evidence_type.reshape(S).astype(jnp.int32)
    e_col = e.reshape(S, 1)

    grid_spec = pltpu.PrefetchScalarGridSpec(
        num_scalar_prefetch=1,
        grid=(S // ROWS_PER_BLOCK,),
        in_specs=[
            pl.BlockSpec((S, 1), lambda i, e_sm: (0, 0)),
            pl.BlockSpec((REL_NUM, HEAD), lambda i, e_sm: (0, 0)),
        ],
        out_specs=pl.BlockSpec(
            (None, ROWS_PER_BLOCK, S, HEAD), lambda i, e_sm: (0, i, 0, 0)
        ),
        scratch_shapes=[pltpu.VMEM((4, S, HEAD), jnp.float32)],
    )

    out = pl.pallas_call(
        _fanout_body,
        grid_spec=grid_spec,
        out_shape=jax.ShapeDtypeStruct((B, S, S, HEAD), jnp.float32),
    )(e, e_col, W)
    return out


# speedup vs baseline: 1.0303x; 1.0285x over previous
import jax
import jax.numpy as jnp
from jax import lax
from jax.experimental import pallas as pl
from jax.experimental.pallas import tpu as pltpu

B = 1
S = 2048
REL_NUM = 16
HEAD = 64
LANE = 128
FOLD = LANE // HEAD
SROWS = S * HEAD // LANE
ROWS_PER_BLOCK = 8


def _fanout_body(e_sm, e2_ref, w_ref, out_ref, slab_ref):
    i = pl.program_id(0)

    @pl.when(i == 0)
    def _build_slabs():
        w = w_ref[...]
        z = jnp.zeros((REL_NUM, HEAD), jnp.float32)
        w2 = jnp.concatenate(
            [
                jnp.concatenate([w, z], axis=1),
                jnp.concatenate([z, w], axis=1),
            ],
            axis=0,
        )
        e_even = e2_ref[:, 0:1]
        e_odd = e2_ref[:, 1:2]
        e_sel = jnp.concatenate(
            [
                jnp.broadcast_to(e_even, (SROWS, REL_NUM)),
                jnp.broadcast_to(e_odd, (SROWS, REL_NUM)),
            ],
            axis=1,
        )
        iota_c = lax.broadcasted_iota(jnp.int32, (SROWS, FOLD * REL_NUM), 1)
        r = lax.rem(iota_c, REL_NUM)
        for a in range(4):
            onehot = (r == (e_sel + 4 * a)).astype(jnp.float32)
            slab_ref[a] = jnp.dot(
                onehot, w2, preferred_element_type=jnp.float32
            )

    for k in range(ROWS_PER_BLOCK):
        a_k = e_sm[i * ROWS_PER_BLOCK + k]
        out_ref[k] = slab_ref[a_k]


@jax.jit
def kernel(evidence_type, W):
    e = evidence_type.reshape(S).astype(jnp.int32)
    e2 = e.reshape(SROWS, FOLD)

    grid_spec = pltpu.PrefetchScalarGridSpec(
        num_scalar_prefetch=1,
        grid=(S // ROWS_PER_BLOCK,),
        in_specs=[
            pl.BlockSpec((SROWS, FOLD), lambda i, e_sm: (0, 0)),
            pl.BlockSpec((REL_NUM, HEAD), lambda i, e_sm: (0, 0)),
        ],
        out_specs=pl.BlockSpec(
            (None, ROWS_PER_BLOCK, SROWS, LANE), lambda i, e_sm: (0, i, 0, 0)
        ),
        scratch_shapes=[pltpu.VMEM((4, SROWS, LANE), jnp.float32)],
    )

    out = pl.pallas_call(
        _fanout_body,
        grid_spec=grid_spec,
        out_shape=jax.ShapeDtypeStruct((B, S, SROWS, LANE), jnp.float32),
    )(e, e2, W)
    return out.reshape(B, S, S, HEAD)


# speedup vs baseline: 6.5085x; 6.3173x over previous
import jax
import jax.numpy as jnp
from jax import lax
from jax.experimental import pallas as pl
from jax.experimental.pallas import tpu as pltpu

B = 1
S = 2048
REL_NUM = 16
HEAD = 64
ROWS_PER_BLOCK = 8


def _fanout_body(e_sm, e_row_ref, w_ref, out_ref, slab_ref):
    i = pl.program_id(0)

    @pl.when(i == 0)
    def _build_slabs():
        w = w_ref[...]
        e_row = e_row_ref[...]
        iota_r = lax.broadcasted_iota(jnp.int32, (REL_NUM, S), 0)
        for a in range(4):
            onehot = (iota_r == (e_row + 4 * a)).astype(jnp.float32)
            slab_ref[a] = lax.dot_general(
                w,
                onehot,
                dimension_numbers=(((0,), (0,)), ((), ())),
                preferred_element_type=jnp.float32,
            )

    for k in range(ROWS_PER_BLOCK):
        a_k = e_sm[i * ROWS_PER_BLOCK + k]
        out_ref[k] = slab_ref[a_k]


@jax.jit
def kernel(evidence_type, W):
    e = evidence_type.reshape(S).astype(jnp.int32)
    e_row = e.reshape(1, S)

    grid_spec = pltpu.PrefetchScalarGridSpec(
        num_scalar_prefetch=1,
        grid=(S // ROWS_PER_BLOCK,),
        in_specs=[
            pl.BlockSpec((1, S), lambda i, e_sm: (0, 0)),
            pl.BlockSpec((REL_NUM, HEAD), lambda i, e_sm: (0, 0)),
        ],
        out_specs=pl.BlockSpec(
            (None, ROWS_PER_BLOCK, HEAD, S), lambda i, e_sm: (0, i, 0, 0)
        ),
        scratch_shapes=[pltpu.VMEM((4, HEAD, S), jnp.float32)],
    )

    out = pl.pallas_call(
        _fanout_body,
        grid_spec=grid_spec,
        out_shape=jax.ShapeDtypeStruct((B, S, HEAD, S), jnp.float32),
    )(e, e_row, W)
    return jnp.transpose(out, (0, 1, 3, 2))
